# baseline (speedup 1.0000x reference)
import functools

import jax
import jax.numpy as jnp
from jax.experimental import pallas as pl
from jax.experimental.pallas import tpu as pltpu


def _rcag_kernel(x_ref, w1_ref, w2_ref, wd_ref, bd_ref, wu_ref, bu_ref,
                 wf_ref, mcol_ref, out_ref, sa_ref, sb_ref, *, H, W, C, nb):
    HW = H * W
    NPI = 8
    NCH = 2 * NPI
    CH = HW // NPI

    pad = (jax.lax.broadcasted_iota(jnp.int32, (8, 2 * HW), 0) == 0
           ).astype(jnp.bfloat16)
    zW = jnp.zeros((C, W), jnp.bfloat16)
    for s_ref in (sa_ref, sb_ref):
        s_ref[3 * C:3 * C + 8, :] = pad
        for base in (0, HW):
            s_ref[0:C, pl.ds(base, W)] = zW
            s_ref[2 * C:3 * C, pl.ds(base + HW - W, W)] = zW

    def store_chunk(s_ref, c, ab):
        base = c * CH
        s_ref[C:2 * C, pl.ds(base, CH)] = ab
        if c % NPI != NPI - 1:
            s_ref[0:C, pl.ds(base + W, CH)] = ab
        else:
            s_ref[0:C, pl.ds(base + W, CH - W)] = ab[:, :CH - W]
        if c % NPI != 0:
            s_ref[2 * C:3 * C, pl.ds(base - W, CH)] = ab
        else:
            s_ref[2 * C:3 * C, pl.ds(base, CH - W)] = ab[:, W:]

    m0 = mcol_ref[0, :, :CH]
    m1 = mcol_ref[1, :, :CH]

    def conv_chunk(s_ref, w_ref, blk, c):
        B = jnp.dot(w_ref[blk], s_ref[:, c * CH:(c + 1) * CH],
                    preferred_element_type=jnp.float32)
        return (B[C:2 * C]
                + pltpu.roll(B[0:C], 1, 1) * m0
                + pltpu.roll(B[2 * C:3 * C], CH - 1, 1) * m1)

    def calayer(y, blk):
        d = jnp.maximum(jnp.sum(wd_ref[blk] * y, axis=0, keepdims=True)
                        + bd_ref[blk], 0.0)
        return jax.nn.sigmoid(jnp.sum(wu_ref[blk] * d, axis=1, keepdims=True)
                              + bu_ref[blk])

    a_ch = [x_ref[c // NPI, :, (c % NPI) * CH:((c % NPI) + 1) * CH]
            for c in range(NCH)]
    for c in range(NCH):
        store_chunk(sa_ref, c, a_ch[c].astype(jnp.bfloat16))

    for blk in range(nb):
        for c in range(NCH):
            r1 = jnp.maximum(conv_chunk(sa_ref, w1_ref, blk, c), 0.0)
            store_chunk(sb_ref, c, r1.astype(jnp.bfloat16))
        r2 = []
        y0 = jnp.zeros((C, 1), jnp.float32)
        y1 = jnp.zeros((C, 1), jnp.float32)
        for c in range(NPI):
            comb = conv_chunk(sb_ref, w2_ref, blk, c)
            r2.append(comb)
            y0 = y0 + jnp.sum(comb, axis=1, keepdims=True)
        s0 = calayer(y0 * (1.0 / HW), blk)
        for c in range(NPI, NCH):
            comb = conv_chunk(sb_ref, w2_ref, blk, c)
            r2.append(comb)
            y1 = y1 + jnp.sum(comb, axis=1, keepdims=True)
            cc = c - NPI
            a_ch[cc] = r2[cc] * s0 + a_ch[cc]
            store_chunk(sa_ref, cc, a_ch[cc].astype(jnp.bfloat16))
        s1 = calayer(y1 * (1.0 / HW), blk)
        for c in range(NPI, NCH):
            a_ch[c] = r2[c] * s1 + a_ch[c]
            store_chunk(sa_ref, c, a_ch[c].astype(jnp.bfloat16))

    for c in range(NCH):
        res = conv_chunk(sa_ref, wf_ref, 0, c)
        lb = (c % NPI) * CH
        out_ref[c // NPI, :, lb:lb + CH] = (
            res + x_ref[c // NPI, :, lb:lb + CH]).astype(out_ref.dtype)


def _stack_weights(w, b, C):
    nb = w.shape[0]
    base = jnp.transpose(w.reshape(nb, 3, 3, C, C),
                         (0, 2, 3, 1, 4)).reshape(nb, 3 * C, 3 * C)
    extra = jnp.zeros((nb, 3 * C, 8), w.dtype)
    extra = extra.at[:, C:2 * C, 0].set(b.reshape(nb, C))
    return jnp.concatenate([base, extra], axis=2).astype(jnp.bfloat16)


def kernel(x, w1, b1, w2, b2, wd, bd, wu, bu, wf, bf):
    N, C, H, W = x.shape
    HW = H * W
    nb = w1.shape[0]
    Cr = wd.shape[-1]

    w1s = _stack_weights(w1, b1, C)
    w2s = _stack_weights(w2, b2, C)
    wfs = _stack_weights(wf, bf.reshape(1, C, 1), C)

    col = jnp.arange(2 * HW, dtype=jnp.int32) % W
    mcol = jnp.stack([(col != 0).astype(jnp.float32),
                      (col != W - 1).astype(jnp.float32)]).reshape(2, 1, 2 * HW)

    kernel_fn = functools.partial(_rcag_kernel, H=H, W=W, C=C, nb=nb)

    def full(shape):
        return pl.BlockSpec(shape, lambda n, _s=shape: (0,) * len(_s))

    out = pl.pallas_call(
        kernel_fn,
        out_shape=jax.ShapeDtypeStruct((N, C, HW), x.dtype),
        grid_spec=pltpu.PrefetchScalarGridSpec(
            num_scalar_prefetch=0,
            grid=(N // 2,),
            in_specs=[
                pl.BlockSpec((2, C, HW), lambda n: (n, 0, 0)),
                full((nb, 3 * C, 3 * C + 8)),
                full((nb, 3 * C, 3 * C + 8)),
                full((nb, C, Cr)), full((nb, 1, Cr)),
                full((nb, C, Cr)), full((nb, C, 1)),
                full((1, 3 * C, 3 * C + 8)),
                full((2, 1, 2 * HW)),
            ],
            out_specs=pl.BlockSpec((2, C, HW), lambda n: (n, 0, 0)),
            scratch_shapes=[pltpu.VMEM((3 * C + 8, 2 * HW), jnp.bfloat16),
                            pltpu.VMEM((3 * C + 8, 2 * HW), jnp.bfloat16)],
        ),
        compiler_params=pltpu.CompilerParams(dimension_semantics=("parallel",)),
    )(x.reshape(N, C, HW),
      w1s, w2s, wd, bd, wu, bu, wfs, mcol)
    return out.reshape(N, C, H, W)


# speedup vs baseline: 1.0642x; 1.0642x over previous
import functools

import jax
import jax.numpy as jnp
from jax.experimental import pallas as pl
from jax.experimental.pallas import tpu as pltpu


def _rcag_kernel(x_ref, w1_ref, w2_ref, wd_ref, bd_ref, wu_ref, bu_ref,
                 wf_ref, mcol_ref, out_ref, s_ref, *, H, W, C, nb):
    HW = H * W
    x = x_ref[0]

    s_ref[0:C, 0:W] = jnp.zeros((C, W), jnp.bfloat16)
    s_ref[2 * C:3 * C, pl.ds(HW - W, W)] = jnp.zeros((C, W), jnp.bfloat16)
    pad = (jax.lax.broadcasted_iota(jnp.int32, (8, HW), 0) == 0
           ).astype(jnp.bfloat16)
    s_ref[3 * C:3 * C + 8, :] = pad

    NCH = 8
    CH = HW // NCH

    def conv3x3(a_bf, w_ref, blk):
        s_ref[0:C, pl.ds(W, HW - W)] = a_bf[:, :HW - W]
        s_ref[C:2 * C, :] = a_bf
        s_ref[2 * C:3 * C, 0:HW - W] = a_bf[:, W:]
        parts = []
        for c in range(NCH):
            B = jnp.dot(w_ref[blk], s_ref[:, c * CH:(c + 1) * CH],
                        preferred_element_type=jnp.float32)
            parts.append(B[C:2 * C]
                         + pltpu.roll(B[0:C], 1, 1) * mcol_ref[0, :, :CH]
                         + pltpu.roll(B[2 * C:3 * C], CH - 1, 1)
                         * mcol_ref[1, :, :CH])
        return jnp.concatenate(parts, axis=1)

    a = x
    for blk in range(nb):
        r = jnp.maximum(conv3x3(a.astype(jnp.bfloat16), w1_ref, blk), 0.0)
        r = conv3x3(r.astype(jnp.bfloat16), w2_ref, blk)
        y = jnp.sum(r, axis=1, keepdims=True) * (1.0 / HW)
        d = jnp.maximum(jnp.sum(wd_ref[blk] * y, axis=0, keepdims=True)
                        + bd_ref[blk], 0.0)
        s = jax.nn.sigmoid(jnp.sum(wu_ref[blk] * d, axis=1, keepdims=True)
                           + bu_ref[blk])
        a = r * s + a

    res = conv3x3(a.astype(jnp.bfloat16), wf_ref, 0)
    out_ref[0] = (res + x).astype(out_ref.dtype)


def _stack_weights(w, b, C):
    nb = w.shape[0]
    base = jnp.transpose(w.reshape(nb, 3, 3, C, C),
                         (0, 2, 3, 1, 4)).reshape(nb, 3 * C, 3 * C)
    extra = jnp.zeros((nb, 3 * C, 8), w.dtype)
    extra = extra.at[:, C:2 * C, 0].set(b.reshape(nb, C))
    return jnp.concatenate([base, extra], axis=2).astype(jnp.bfloat16)


def kernel(x, w1, b1, w2, b2, wd, bd, wu, bu, wf, bf):
    N, C, H, W = x.shape
    HW = H * W
    nb = w1.shape[0]
    Cr = wd.shape[-1]

    w1s = _stack_weights(w1, b1, C)
    w2s = _stack_weights(w2, b2, C)
    wfs = _stack_weights(wf, bf.reshape(1, C, 1), C)

    col = jnp.arange(HW, dtype=jnp.int32) % W
    mcol = jnp.stack([(col != 0).astype(jnp.float32),
                      (col != W - 1).astype(jnp.float32)]).reshape(2, 1, HW)

    kernel_fn = functools.partial(_rcag_kernel, H=H, W=W, C=C, nb=nb)

    def full(shape):
        return pl.BlockSpec(shape, lambda n, _s=shape: (0,) * len(_s))

    out = pl.pallas_call(
        kernel_fn,
        out_shape=jax.ShapeDtypeStruct((N, C, HW), x.dtype),
        grid_spec=pltpu.PrefetchScalarGridSpec(
            num_scalar_prefetch=0,
            grid=(N,),
            in_specs=[
                pl.BlockSpec((1, C, HW), lambda n: (n, 0, 0)),
                full((nb, 3 * C, 3 * C + 8)),
                full((nb, 3 * C, 3 * C + 8)),
                full((nb, C, Cr)), full((nb, 1, Cr)),
                full((nb, C, Cr)), full((nb, C, 1)),
                full((1, 3 * C, 3 * C + 8)),
                full((2, 1, HW)),
            ],
            out_specs=pl.BlockSpec((1, C, HW), lambda n: (n, 0, 0)),
            scratch_shapes=[pltpu.VMEM((3 * C + 8, HW), jnp.bfloat16)],
        ),
        compiler_params=pltpu.CompilerParams(dimension_semantics=("parallel",)),
    )(x.reshape(N, C, HW),
      w1s, w2s, wd, bd, wu, bu, wfs, mcol)
    return out.reshape(N, C, H, W)
